# Initial kernel scaffold; baseline (speedup 1.0000x reference)
import jax
import jax.numpy as jnp
from jax.experimental import pallas as pl


def kernel(x, W, b, gamma, beta, ema_load, top_k):
    raise NotImplementedError("write your pallas kernel here")


# speedup vs baseline: 3.1065x; 3.1065x over previous
import functools

import jax
import jax.numpy as jnp
from jax.experimental import pallas as pl

D_MODEL = 2048
N_EXP = 16
TOK_BLK = 512


def _router_kernel(x_ref, wt_ref, b_ref, gamma_ref, beta_ref, ema_ref,
                   sparse_ref, idx_ref, acc_ref, lb_ref, *, n_tokens):
    step = pl.program_id(0)
    n_steps = pl.num_programs(0)

    xb = x_ref[...]
    mu = jnp.mean(xb, axis=1, keepdims=True)
    xc = xb - mu
    var = jnp.mean(xc * xc, axis=1, keepdims=True)
    xn = xc * jax.lax.rsqrt(var + 1e-5) * gamma_ref[...] + beta_ref[...]

    logits = jnp.dot(xn, wt_ref[...], preferred_element_type=jnp.float32)
    logits = logits + b_ref[...]

    m = jnp.max(logits, axis=1, keepdims=True)
    e = jnp.exp(logits - m)
    w = e / jnp.sum(e, axis=1, keepdims=True)

    iota = jax.lax.broadcasted_iota(jnp.int32, w.shape, 1)
    big = jnp.int32(N_EXP)

    v1 = jnp.max(w, axis=1, keepdims=True)
    a1 = jnp.min(jnp.where(w == v1, iota, big), axis=1, keepdims=True)
    w2 = jnp.where(iota == a1, -jnp.inf, w)
    v2 = jnp.max(w2, axis=1, keepdims=True)
    a2 = jnp.min(jnp.where((w == v2) & (iota != a1), iota, big),
                 axis=1, keepdims=True)

    denom = v1 + v2 + 1e-8
    sparse = jnp.where(iota == a1, v1, 0.0) + jnp.where(iota == a2, v2, 0.0)
    sparse_ref[...] = sparse / denom
    idx_ref[...] = jnp.concatenate([a1, a2], axis=1)

    @pl.when(step == 0)
    def _init():
        acc_ref[...] = jnp.zeros_like(acc_ref)

    acc_ref[...] += jnp.sum(w, axis=0, keepdims=True)

    @pl.when(step == n_steps - 1)
    def _finish():
        mean_w = acc_ref[...] / n_tokens
        lb = jnp.sum(mean_w * jnp.log(mean_w + 1e-8))
        uniform = 1.0 / N_EXP
        threshold = uniform + min(0.15, (1.0 - uniform) * 0.3)
        penalty = jnp.maximum(jnp.max(ema_ref[...]) - threshold, 0.0)
        lb_ref[...] = jnp.reshape(lb + 0.1 * penalty, (1, 1))


def kernel(x, W, b, gamma, beta, ema_load, top_k):
    B, T, D = x.shape
    n_tokens = B * T
    x2 = x.reshape(n_tokens, D)
    wt = W.T
    grid = (n_tokens // TOK_BLK,)

    out_shapes = (
        jax.ShapeDtypeStruct((n_tokens, N_EXP), jnp.float32),
        jax.ShapeDtypeStruct((n_tokens, 2), jnp.int32),
        jax.ShapeDtypeStruct((1, N_EXP), jnp.float32),
        jax.ShapeDtypeStruct((1, 1), jnp.float32),
    )
    const_spec = lambda shape: pl.BlockSpec(shape, lambda i: (0, 0))

    sparse, idx, _, lb = pl.pallas_call(
        functools.partial(_router_kernel, n_tokens=n_tokens),
        grid=grid,
        in_specs=[
            pl.BlockSpec((TOK_BLK, D), lambda i: (i, 0)),
            const_spec((D, N_EXP)),
            const_spec((1, N_EXP)),
            const_spec((1, D)),
            const_spec((1, D)),
            const_spec((1, N_EXP)),
        ],
        out_specs=(
            pl.BlockSpec((TOK_BLK, N_EXP), lambda i: (i, 0)),
            pl.BlockSpec((TOK_BLK, 2), lambda i: (i, 0)),
            const_spec((1, N_EXP)),
            const_spec((1, 1)),
        ),
        out_shape=out_shapes,
    )(x2, wt, b.reshape(1, N_EXP), gamma.reshape(1, D), beta.reshape(1, D),
      ema_load.reshape(1, N_EXP))

    sparse_w = sparse.reshape(B, T, N_EXP)
    indices = idx.reshape(B, T, 2)
    lb_loss = lb[0, 0]
    return (sparse_w, indices, lb_loss)
